# baseline (speedup 1.0000x reference)
import jax
import jax.numpy as jnp
from jax import lax
from jax.experimental import pallas as pl
from jax.experimental.pallas import tpu as pltpu
from jax.experimental.pallas import tpu_sc as plsc

N_TOTAL = 1179648
_info = plsc.get_sparse_core_info()
NC, NS, L = _info.num_cores, _info.num_subcores, _info.num_lanes
NW = NC * NS
B_PER_W = N_TOTAL // NW


CHUNKS = (2304, 15360, 14592, 4608)
OFFS = (0, 2304, 17664, 32256)
NCH = len(CHUNKS)


def _body(w_hbm, idx_hbm, mask_hbm, out_hbm, idx_v, rows_v, mask_v,
          isem, gsem, osem):
    wid = lax.axis_index("s") * NC + lax.axis_index("c")
    base = wid * B_PER_W

    def gather(c):
        return pltpu.async_copy(
            w_hbm.at[idx_v.at[pl.ds(OFFS[c], CHUNKS[c])]],
            rows_v.at[pl.ds(OFFS[c], CHUNKS[c])],
            gsem,
        )

    idx_cps = [
        pltpu.async_copy(
            idx_hbm.at[pl.ds(base + OFFS[c], CHUNKS[c])],
            idx_v.at[pl.ds(OFFS[c], CHUNKS[c])],
            isem,
        )
        for c in range(NCH)
    ]
    mask_cp = pltpu.async_copy(mask_hbm.at[pl.ds(base, B_PER_W)], mask_v,
                               osem)
    idx_cps[0].wait()
    gathers = [gather(0)]

    writes = []
    for c in range(NCH):
        if c + 1 < NCH:
            idx_cps[c + 1].wait()
            gathers.append(gather(c + 1))
        if c == 0:
            mask_cp.wait()
        gathers[c].wait()

        def mul(i, carry):
            sl = pl.ds(OFFS[c] + i * L, L)
            rows_v[sl] = rows_v[sl] * mask_v[sl]
            return carry

        lax.fori_loop(0, CHUNKS[c] // L, mul, 0, unroll=8)
        writes.append(
            pltpu.async_copy(
                rows_v.at[pl.ds(OFFS[c], CHUNKS[c])],
                out_hbm.at[pl.ds(base + OFFS[c], CHUNKS[c])],
                osem,
            )
        )
    for wcp in writes:
        wcp.wait()


@jax.jit
def _super_params(w, perm_key, mask_key):
    mesh = plsc.VectorSubcoreMesh(core_axis_name="c", subcore_axis_name="s")
    return pl.kernel(
        _body,
        out_type=jax.ShapeDtypeStruct((N_TOTAL,), jnp.float32),
        mesh=mesh,
        scratch_types=[
            pltpu.VMEM((B_PER_W,), jnp.int32),
            pltpu.VMEM((B_PER_W,), jnp.float32),
            pltpu.VMEM((B_PER_W,), jnp.float32),
            pltpu.SemaphoreType.DMA,
            pltpu.SemaphoreType.DMA,
            pltpu.SemaphoreType.DMA,
        ],
    )(w, perm_key, mask_key)


def kernel(w, perm_key, mask_key):
    return _super_params(w, perm_key.astype(jnp.int32), mask_key)


# speedup vs baseline: 1.0035x; 1.0035x over previous
import jax
import jax.numpy as jnp
from jax import lax
from jax.experimental import pallas as pl
from jax.experimental.pallas import tpu as pltpu
from jax.experimental.pallas import tpu_sc as plsc

N_TOTAL = 1179648
_info = plsc.get_sparse_core_info()
NC, NS, L = _info.num_cores, _info.num_subcores, _info.num_lanes
NW = NC * NS
B_PER_W = N_TOTAL // NW


CHUNKS = (3072, 15360, 14592, 3840)
OFFS = (0, 3072, 18432, 33024)
NCH = len(CHUNKS)


def _body(w_hbm, idx_hbm, mask_hbm, out_hbm, idx_v, rows_v, mask_v,
          isem, gsem, osem):
    wid = lax.axis_index("s") * NC + lax.axis_index("c")
    base = wid * B_PER_W

    def gather(c):
        return pltpu.async_copy(
            w_hbm.at[idx_v.at[pl.ds(OFFS[c], CHUNKS[c])]],
            rows_v.at[pl.ds(OFFS[c], CHUNKS[c])],
            gsem,
        )

    idx_cps = [
        pltpu.async_copy(
            idx_hbm.at[pl.ds(base + OFFS[c], CHUNKS[c])],
            idx_v.at[pl.ds(OFFS[c], CHUNKS[c])],
            isem,
        )
        for c in range(NCH)
    ]
    mask_cp = pltpu.async_copy(mask_hbm.at[pl.ds(base, B_PER_W)], mask_v,
                               osem)
    idx_cps[0].wait()
    gathers = [gather(0)]

    writes = []
    for c in range(NCH):
        if c + 1 < NCH:
            idx_cps[c + 1].wait()
            gathers.append(gather(c + 1))
        if c == 0:
            mask_cp.wait()
        gathers[c].wait()

        def mul(i, carry):
            sl = pl.ds(OFFS[c] + i * L, L)
            rows_v[sl] = rows_v[sl] * mask_v[sl]
            return carry

        lax.fori_loop(0, CHUNKS[c] // L, mul, 0, unroll=8)
        writes.append(
            pltpu.async_copy(
                rows_v.at[pl.ds(OFFS[c], CHUNKS[c])],
                out_hbm.at[pl.ds(base + OFFS[c], CHUNKS[c])],
                osem,
            )
        )
    for wcp in writes:
        wcp.wait()


@jax.jit
def _super_params(w, perm_key, mask_key):
    mesh = plsc.VectorSubcoreMesh(core_axis_name="c", subcore_axis_name="s")
    return pl.kernel(
        _body,
        out_type=jax.ShapeDtypeStruct((N_TOTAL,), jnp.float32),
        mesh=mesh,
        scratch_types=[
            pltpu.VMEM((B_PER_W,), jnp.int32),
            pltpu.VMEM((B_PER_W,), jnp.float32),
            pltpu.VMEM((B_PER_W,), jnp.float32),
            pltpu.SemaphoreType.DMA,
            pltpu.SemaphoreType.DMA,
            pltpu.SemaphoreType.DMA,
        ],
    )(w, perm_key, mask_key)


def kernel(w, perm_key, mask_key):
    return _super_params(w, perm_key.astype(jnp.int32), mask_key)


# speedup vs baseline: 1.0061x; 1.0025x over previous
import jax
import jax.numpy as jnp
from jax import lax
from jax.experimental import pallas as pl
from jax.experimental.pallas import tpu as pltpu
from jax.experimental.pallas import tpu_sc as plsc

N_TOTAL = 1179648
_info = plsc.get_sparse_core_info()
NC, NS, L = _info.num_cores, _info.num_subcores, _info.num_lanes
NW = NC * NS
B_PER_W = N_TOTAL // NW


CHUNKS = (3072, 14592, 14592, 4608)
OFFS = (0, 3072, 17664, 32256)
NCH = len(CHUNKS)


def _body(w_hbm, idx_hbm, mask_hbm, out_hbm, idx_v, rows_v, mask_v,
          isem, gsem, osem):
    wid = lax.axis_index("s") * NC + lax.axis_index("c")
    base = wid * B_PER_W

    def gather(c):
        return pltpu.async_copy(
            w_hbm.at[idx_v.at[pl.ds(OFFS[c], CHUNKS[c])]],
            rows_v.at[pl.ds(OFFS[c], CHUNKS[c])],
            gsem,
        )

    idx_cps = [
        pltpu.async_copy(
            idx_hbm.at[pl.ds(base + OFFS[c], CHUNKS[c])],
            idx_v.at[pl.ds(OFFS[c], CHUNKS[c])],
            isem,
        )
        for c in range(NCH)
    ]
    mask_cp = pltpu.async_copy(mask_hbm.at[pl.ds(base, B_PER_W)], mask_v,
                               osem)
    idx_cps[0].wait()
    gathers = [gather(0)]

    writes = []
    for c in range(NCH):
        if c + 1 < NCH:
            idx_cps[c + 1].wait()
            gathers.append(gather(c + 1))
        if c == 0:
            mask_cp.wait()
        gathers[c].wait()

        def mul(i, carry):
            sl = pl.ds(OFFS[c] + i * L, L)
            rows_v[sl] = rows_v[sl] * mask_v[sl]
            return carry

        lax.fori_loop(0, CHUNKS[c] // L, mul, 0, unroll=8)
        writes.append(
            pltpu.async_copy(
                rows_v.at[pl.ds(OFFS[c], CHUNKS[c])],
                out_hbm.at[pl.ds(base + OFFS[c], CHUNKS[c])],
                osem,
            )
        )
    for wcp in writes:
        wcp.wait()


@jax.jit
def _super_params(w, perm_key, mask_key):
    mesh = plsc.VectorSubcoreMesh(core_axis_name="c", subcore_axis_name="s")
    return pl.kernel(
        _body,
        out_type=jax.ShapeDtypeStruct((N_TOTAL,), jnp.float32),
        mesh=mesh,
        scratch_types=[
            pltpu.VMEM((B_PER_W,), jnp.int32),
            pltpu.VMEM((B_PER_W,), jnp.float32),
            pltpu.VMEM((B_PER_W,), jnp.float32),
            pltpu.SemaphoreType.DMA,
            pltpu.SemaphoreType.DMA,
            pltpu.SemaphoreType.DMA,
        ],
    )(w, perm_key, mask_key)


def kernel(w, perm_key, mask_key):
    return _super_params(w, perm_key.astype(jnp.int32), mask_key)
